# Initial kernel scaffold; baseline (speedup 1.0000x reference)
import jax
import jax.numpy as jnp
from jax.experimental import pallas as pl


def kernel(seed_points, seed_feats, mem, in_proj_w, in_proj_b, out_w, out_b, bn0_g, bn0_b, w1, b1, bn1_g, bn1_b, w2, b2, bn2_g, bn2_b, vote_w, vote_b):
    raise NotImplementedError("write your pallas kernel here")


# speedup vs baseline: 2.3066x; 2.3066x over previous
import functools

import jax
import jax.numpy as jnp
from jax import lax
from jax.experimental import pallas as pl
from jax.experimental.pallas import tpu as pltpu

_NHEAD = 4


def _fused_body(
    feats_ref, spt_ref, mem_ref,
    wq_ref, wkT_ref, wvT_ref, bq_ref, bk_ref, bv_ref,
    outw_ref, outb_ref,
    g0_ref, be0_ref, w1_ref, b1_ref, g1_ref, be1_ref,
    w2_ref, b2_ref, g2_ref, be2_ref,
    vwr_ref, vbr_ref, vwo_ref, vbo_ref,
    feats_out, vp_out, off_out,
    xs_ref, acc_ref,
    *, B, C, N,
):
    s = pl.program_id(0)
    b = pl.program_id(1)
    inv_bn = 1.0 / (B * N)
    bf16 = jnp.bfloat16
    dh = C // _NHEAD
    scale = dh ** -0.5

    def accum(k, y):
        s1 = jnp.sum(y, axis=1, keepdims=True)
        s2 = jnp.sum(y * y, axis=1, keepdims=True)

        @pl.when(b == 0)
        def _():
            acc_ref[k, 0] = s1
            acc_ref[k, 1] = s2

        @pl.when(b > 0)
        def _():
            acc_ref[k, 0] += s1
            acc_ref[k, 1] += s2

    def bn(k, y, g_ref, be_ref):
        m = acc_ref[k, 0] * inv_bn
        v = acc_ref[k, 1] * inv_bn - m * m
        return (y - m) * lax.rsqrt(v + 1e-5) * g_ref[...] + be_ref[...]

    def mm(a, x):
        return jnp.dot(a, x, preferred_element_type=jnp.float32)

    @pl.when(s == 0)
    def _stage0():
        x = feats_ref[0]
        xb = x.astype(bf16)
        qp = mm(wq_ref[...], xb) + bq_ref[...]
        memb = mem_ref[...].astype(bf16)
        kp = mm(memb, wkT_ref[...]) + bk_ref[...]
        vp = mm(memb, wvT_ref[...]) + bv_ref[...]
        ohs = []
        for h in range(_NHEAD):
            lo, hi = h * dh, (h + 1) * dh
            qh = (qp[lo:hi, :] * scale).astype(bf16)
            kh = kp[:, lo:hi].astype(bf16)
            sc = mm(kh, qh)
            mx = jnp.max(sc, axis=0, keepdims=True)
            e = jnp.exp(sc - mx)
            den = jnp.sum(e, axis=0, keepdims=True)
            a = (e / den).astype(bf16)
            vh = vp[:, lo:hi].astype(bf16)
            oh = lax.dot_general(
                vh, a, (((0,), (0,)), ((), ())),
                preferred_element_type=jnp.float32)
            ohs.append(oh)
        o = jnp.concatenate(ohs, axis=0).astype(bf16)
        x0 = x + mm(outw_ref[...], o) + outb_ref[...]
        accum(0, x0)
        xs_ref[b] = x0

    @pl.when(s == 1)
    def _stage1():
        xn = bn(0, xs_ref[b], g0_ref, be0_ref)
        y1 = mm(w1_ref[...], xn.astype(bf16)) + b1_ref[...]
        accum(1, y1)
        xs_ref[b] = y1

    @pl.when(s == 2)
    def _stage2():
        h1 = jnp.maximum(bn(1, xs_ref[b], g1_ref, be1_ref), 0.0)
        y2 = mm(w2_ref[...], h1.astype(bf16)) + b2_ref[...]
        accum(2, y2)
        xs_ref[b] = y2

    @pl.when(s == 3)
    def _stage3():
        h2 = jnp.maximum(bn(2, xs_ref[b], g2_ref, be2_ref), 0.0).astype(bf16)
        res = mm(vwr_ref[...], h2) + vbr_ref[...]
        off8 = mm(vwo_ref[...], h2)
        off = off8[0:3, :] + vbo_ref[...]
        vp_out[0] = spt_ref[0] + off
        off_out[0] = off
        f = feats_ref[0] + res
        nrm = jnp.sqrt(jnp.sum(f * f, axis=0, keepdims=True))
        feats_out[0] = f / nrm


def kernel(seed_points, seed_feats, mem, in_proj_w, in_proj_b, out_w, out_b,
           bn0_g, bn0_b, w1, b1, bn1_g, bn1_b, w2, b2, bn2_g, bn2_b,
           vote_w, vote_b):
    B, C, N = seed_feats.shape
    S = mem.shape[0]
    bf16 = jnp.bfloat16
    f32 = jnp.float32

    spt = seed_points.transpose(0, 2, 1)
    wq = in_proj_w[:C].astype(bf16)
    wkT = in_proj_w[C:2 * C].T.astype(bf16)
    wvT = in_proj_w[2 * C:].T.astype(bf16)
    bq = in_proj_b[:C].reshape(C, 1)
    bk = in_proj_b[C:2 * C].reshape(1, C)
    bv = in_proj_b[2 * C:].reshape(1, C)
    vwr = vote_w[3:].astype(bf16)
    vwo = jnp.zeros((8, C), bf16).at[:3].set(vote_w[:3].astype(bf16))
    vbr = vote_b[3:].reshape(C, 1)
    vbo = vote_b[:3].reshape(3, 1)

    col = lambda v: v.reshape(C, 1)

    def full(shape):
        return pl.BlockSpec(shape, lambda s, b: (0,) * len(shape))

    feats_spec = pl.BlockSpec(
        (1, C, N),
        lambda s, b: (jnp.where((s == 0) | (s == 3), b, B - 1), 0, 0))
    spt_spec = pl.BlockSpec(
        (1, 3, N), lambda s, b: (jnp.where(s == 3, b, 0), 0, 0))
    out_idx = lambda s, b: (jnp.where(s == 3, b, 0), 0, 0)

    feats_out, vp_t, off_t = pl.pallas_call(
        functools.partial(_fused_body, B=B, C=C, N=N),
        grid=(4, B),
        in_specs=[
            feats_spec, spt_spec, full((S, C)),
            full((C, C)), full((C, C)), full((C, C)),
            full((C, 1)), full((1, C)), full((1, C)),
            full((C, C)), full((C, 1)),
            full((C, 1)), full((C, 1)), full((C, C)), full((C, 1)),
            full((C, 1)), full((C, 1)),
            full((C, C)), full((C, 1)), full((C, 1)), full((C, 1)),
            full((C, C)), full((C, 1)), full((8, C)), full((3, 1)),
        ],
        out_specs=[
            pl.BlockSpec((1, C, N), out_idx),
            pl.BlockSpec((1, 3, N), out_idx),
            pl.BlockSpec((1, 3, N), out_idx),
        ],
        out_shape=[
            jax.ShapeDtypeStruct((B, C, N), f32),
            jax.ShapeDtypeStruct((B, 3, N), f32),
            jax.ShapeDtypeStruct((B, 3, N), f32),
        ],
        scratch_shapes=[
            pltpu.VMEM((B, C, N), f32),
            pltpu.VMEM((3, 2, C, 1), f32),
        ],
    )(
        seed_feats, spt, mem,
        wq, wkT, wvT, bq, bk, bv,
        out_w.astype(bf16), col(out_b),
        col(bn0_g), col(bn0_b), w1.astype(bf16), col(b1),
        col(bn1_g), col(bn1_b),
        w2.astype(bf16), col(b2), col(bn2_g), col(bn2_b),
        vwr, vbr, vwo, vbo,
    )

    vote_points = vp_t.transpose(0, 2, 1)
    offset = off_t.transpose(0, 2, 1)
    return vote_points, feats_out, offset
